# baseline (speedup 1.0000x reference)
import functools

import jax
import jax.numpy as jnp
from jax import lax
from jax.experimental import pallas as pl
from jax.experimental.pallas import tpu as pltpu
from jax.experimental.pallas import tpu_sc as plsc

N = 10000
E = 160000
D = 256
Q = 64
TILES = 16
GB = 320
NGRP = 32
EPT = NGRP * GB
E_PAD = TILES * EPT
ACC_ROWS = 10240
ROWS_PT = ACC_ROWS // TILES
LAST_ROWS = N - (TILES - 1) * ROWS_PT


def _sc_body(xq0, xq1, xq2, xq3, src_hbm, dst_hbm,
             sq0, sq1, sq2, sq3, deg32_hbm,
             src_v, dst_v, bufs, deg_local,
             acc,
             sem_g, sem_s):
    c = lax.axis_index("c")
    s = lax.axis_index("s")

    pltpu.sync_copy(src_hbm.at[s], src_v)
    pltpu.sync_copy(dst_hbm.at[s], dst_v)

    r0 = s * ROWS_PT

    def zero_deg(i, carry):
        deg_local[pl.ds(i * 16, 16)] = jnp.zeros((16,), jnp.float32)
        return carry

    lax.fori_loop(0, ACC_ROWS // 16, zero_deg, 0)

    def phase(x_hbm, s_hbm, with_deg):
        @pl.when(s < TILES - 1)
        def _():
            pltpu.sync_copy(x_hbm.at[pl.ds(r0, ROWS_PT)],
                            acc.at[pl.ds(r0, ROWS_PT)])

        @pl.when(s == TILES - 1)
        def _():
            pltpu.sync_copy(x_hbm.at[pl.ds((TILES - 1) * ROWS_PT, LAST_ROWS)],
                            acc.at[pl.ds((TILES - 1) * ROWS_PT, LAST_ROWS)])

        plsc.subcore_barrier()

        def gather_start(g, p):
            pltpu.make_async_copy(x_hbm.at[src_v.at[g]], bufs.at[p],
                                  sem_g).start()

        def gather_wait():
            pltpu.make_async_copy(x_hbm.at[pl.ds(0, GB)], bufs.at[0],
                                  sem_g).wait()

        def scatter_start(g, p):
            pltpu.async_copy(bufs.at[p], acc.at[dst_v.at[g]], sem_s, add=True)

        def scatter_wait():
            pltpu.make_async_copy(bufs.at[0], acc.at[pl.ds(0, GB)],
                                  sem_s).wait()

        if with_deg:
            deg_lo = c * (NGRP // 2)
            deg_hi = deg_lo + NGRP // 2

        def count_deg(g):
            @pl.when((g >= deg_lo) & (g < deg_hi))
            def _():
                ones16 = jnp.ones((16,), jnp.float32)
                for k in range(GB // 16):
                    idx = dst_v[g, pl.ds(k * 16, 16)]
                    plsc.addupdate_scatter(deg_local, [idx], ones16)

        gather_start(0, 0)

        def blk(g, carry):
            gather_wait()
            scatter_start(g, lax.rem(g, 2))
            if with_deg:
                count_deg(g)

            @pl.when(g >= 1)
            def _():
                scatter_wait()

            gather_start(g + 1, lax.rem(g + 1, 2))
            return carry

        lax.fori_loop(0, NGRP - 1, blk, 0)
        gather_wait()
        scatter_start(NGRP - 1, lax.rem(NGRP - 1, 2))
        if with_deg:
            count_deg(NGRP - 1)
        scatter_wait()
        scatter_wait()

        plsc.subcore_barrier()

        pltpu.sync_copy(acc.at[pl.ds(r0, ROWS_PT)], s_hbm.at[pl.ds(r0, ROWS_PT)])
        if with_deg:
            pltpu.sync_copy(deg_local, deg32_hbm.at[c * TILES + s])
        plsc.subcore_barrier()

    @pl.when(c == 0)
    def _():
        phase(xq0, sq0, True)
        phase(xq1, sq1, False)

    @pl.when(c == 1)
    def _():
        phase(xq2, sq2, True)
        phase(xq3, sq3, False)


_sc_agg = functools.partial(
    pl.kernel,
    out_type=(
        jax.ShapeDtypeStruct((ACC_ROWS, Q), jnp.float32),
        jax.ShapeDtypeStruct((ACC_ROWS, Q), jnp.float32),
        jax.ShapeDtypeStruct((ACC_ROWS, Q), jnp.float32),
        jax.ShapeDtypeStruct((ACC_ROWS, Q), jnp.float32),
        jax.ShapeDtypeStruct((2 * TILES, ACC_ROWS), jnp.float32),
    ),
    mesh=plsc.VectorSubcoreMesh(core_axis_name="c", subcore_axis_name="s"),
    compiler_params=pltpu.CompilerParams(use_tc_tiling_on_sc=False,
                                         needs_layout_passes=False),
    scratch_types=[
        pltpu.VMEM((NGRP, GB), jnp.int32),
        pltpu.VMEM((NGRP, GB), jnp.int32),
        pltpu.VMEM((2, GB, Q), jnp.float32),
        pltpu.VMEM((ACC_ROWS,), jnp.float32),
        pltpu.VMEM_SHARED((ACC_ROWS, Q), jnp.float32),
        pltpu.SemaphoreType.DMA,
        pltpu.SemaphoreType.DMA,
    ],
)(_sc_body)


M_BLK = 1000


def _tc_body(s0_ref, s1_ref, s2_ref, s3_ref, deg_ref,
             w0_ref, w1_ref, w2_ref, w3_ref, b_ref, out_ref):
    acc = jnp.dot(s0_ref[...], w0_ref[...], preferred_element_type=jnp.float32)
    acc = acc + jnp.dot(s1_ref[...], w1_ref[...], preferred_element_type=jnp.float32)
    acc = acc + jnp.dot(s2_ref[...], w2_ref[...], preferred_element_type=jnp.float32)
    acc = acc + jnp.dot(s3_ref[...], w3_ref[...], preferred_element_type=jnp.float32)
    deg = jnp.maximum(jnp.sum(deg_ref[...], axis=1, keepdims=True), 1.0)
    out_ref[...] = acc / deg + b_ref[...]


_tc_linear = pl.pallas_call(
    _tc_body,
    grid=(N // M_BLK,),
    in_specs=[
        pl.BlockSpec((M_BLK, Q), lambda i: (i, 0)),
        pl.BlockSpec((M_BLK, Q), lambda i: (i, 0)),
        pl.BlockSpec((M_BLK, Q), lambda i: (i, 0)),
        pl.BlockSpec((M_BLK, Q), lambda i: (i, 0)),
        pl.BlockSpec((M_BLK, 2 * TILES), lambda i: (i, 0)),
        pl.BlockSpec((Q, D), lambda i: (0, 0)),
        pl.BlockSpec((Q, D), lambda i: (0, 0)),
        pl.BlockSpec((Q, D), lambda i: (0, 0)),
        pl.BlockSpec((Q, D), lambda i: (0, 0)),
        pl.BlockSpec((1, D), lambda i: (0, 0)),
    ],
    out_specs=pl.BlockSpec((M_BLK, D), lambda i: (i, 0)),
    out_shape=jax.ShapeDtypeStruct((N, D), jnp.float32),
)


def kernel(x, edge_index, W, b):
    src = edge_index[0]
    dst = edge_index[1]
    pad = E_PAD - E
    srcp = jnp.concatenate([src, jnp.zeros((pad,), jnp.int32)]).reshape(
        TILES, NGRP, GB)
    dstp = jnp.concatenate([dst, jnp.full((pad,), N, jnp.int32)]).reshape(
        TILES, NGRP, GB)
    s0, s1, s2, s3, deg32 = _sc_agg(x[:, :Q], x[:, Q:2 * Q],
                                    x[:, 2 * Q:3 * Q], x[:, 3 * Q:],
                                    srcp, dstp)
    wq = [W[:, q * Q:(q + 1) * Q].T for q in range(4)]
    return _tc_linear(s0, s1, s2, s3, deg32.T,
                      wq[0], wq[1], wq[2], wq[3], b.reshape(1, D))
